# baseline (speedup 1.0000x reference)
import functools

import jax
import jax.numpy as jnp
from jax import lax
from jax.experimental import pallas as pl
from jax.experimental.pallas import tpu as pltpu
from jax.experimental.pallas import tpu_sc as plsc

_B, _S, _D = 4, 8192, 1024
_NC, _NS, _L = 2, 16, 16
_NW = _NC * _NS
_C = 32
_U = 8
_ROWS_PER_W = _S // _NW
_CHUNK_W = _C * _D


def _sc_body(x_hbm, p_hbm, o_hbm, p_v, x_v):
    wid = lax.axis_index("s") * _NC + lax.axis_index("c")
    s_base = wid * _ROWS_PER_W

    def chunk_body(ci, carry):
        s0 = s_base + ci * _C
        pltpu.sync_copy(p_hbm.at[pl.ds(s0 * _D, _CHUNK_W)], p_v)

        def batch_body(b, carry2):
            r0 = (b * _S + s0) * _D
            pltpu.sync_copy(x_hbm.at[pl.ds(r0, _CHUNK_W)], x_v)

            def add_body(k, carry3):
                for u in range(_U):
                    off = (k * _U + u) * _L
                    plsc.addupdate(x_v.at[pl.ds(off, _L)], p_v[pl.ds(off, _L)])
                return carry3

            lax.fori_loop(0, _CHUNK_W // (_L * _U), add_body, 0)
            pltpu.sync_copy(x_v, o_hbm.at[pl.ds(r0, _CHUNK_W)])
            return carry2

        lax.fori_loop(0, _B, batch_body, 0)
        return carry

    lax.fori_loop(0, _ROWS_PER_W // _C, chunk_body, 0)


@functools.partial(
    pl.kernel,
    out_type=jax.ShapeDtypeStruct((_B * _S * _D,), jnp.float32),
    mesh=plsc.VectorSubcoreMesh(core_axis_name="c", subcore_axis_name="s"),
    scratch_types=[
        pltpu.VMEM((_CHUNK_W,), jnp.float32),
        pltpu.VMEM((_CHUNK_W,), jnp.float32),
    ],
)
def _sc_add(x_hbm, p_hbm, o_hbm, p_v, x_v):
    _sc_body(x_hbm, p_hbm, o_hbm, p_v, x_v)


def kernel(x, pos_table):
    B, S, D = x.shape
    out = _sc_add(x.reshape(B * S * D), pos_table.reshape(S * D))
    return out.reshape(B, S, D)


# speedup vs baseline: 5.1593x; 5.1593x over previous
import jax
import jax.numpy as jnp
from jax.experimental import pallas as pl


_TS = 2048


def _add_kernel(x_ref, p_ref, o_ref):
    o_ref[...] = x_ref[...] + p_ref[...]


def kernel(x, pos_table):
    B, S, D = x.shape
    ts = _TS
    grid = (S // ts, B)
    return pl.pallas_call(
        _add_kernel,
        grid=grid,
        in_specs=[
            pl.BlockSpec((1, ts, D), lambda s, b: (b, s, 0)),
            pl.BlockSpec((ts, D), lambda s, b: (s, 0)),
        ],
        out_specs=pl.BlockSpec((1, ts, D), lambda s, b: (b, s, 0)),
        out_shape=jax.ShapeDtypeStruct((B, S, D), x.dtype),
    )(x, pos_table)
